# baseline (speedup 1.0000x reference)
import functools

import jax
import jax.numpy as jnp
from jax import lax
from jax.experimental import pallas as pl
from jax.experimental.pallas import tpu as pltpu
from jax.experimental.pallas import tpu_sc as plsc

N = 100000
G = 5000
NC = 2
NS = 16
L = 16

C1 = 6272
C2 = 3136
GPAD = 5120
ZB = GPAD // NS

_mesh = plsc.VectorSubcoreMesh(
    core_axis_name="c", subcore_axis_name="s", num_cores=NC, num_subcores=NS)


@functools.partial(
    pl.kernel,
    out_type=jax.ShapeDtypeStruct((N,), jnp.float32),
    mesh=_mesh,
    scratch_types=[
        pltpu.VMEM((C1,), jnp.float32),
        pltpu.VMEM((C1,), jnp.float32),
        pltpu.VMEM((C1 + L,), jnp.int32),
        pltpu.VMEM((ZB,), jnp.float32),
        pltpu.VMEM((GPAD,), jnp.int32),
        pltpu.VMEM((C2,), jnp.float32),
        pltpu.VMEM((C2,), jnp.float32),
        pltpu.VMEM((C2,), jnp.int32),
        pltpu.VMEM((C2,), jnp.float32),
        pltpu.VMEM((GPAD,), jnp.float32),
        pltpu.VMEM((GPAD,), jnp.float32),
        pltpu.VMEM_SHARED((GPAD,), jnp.float32),
        pltpu.VMEM_SHARED((GPAD,), jnp.float32),
        pltpu.SemaphoreType.DMA,
        pltpu.SemaphoreType.DMA,
    ],
    compiler_params=pltpu.CompilerParams(needs_layout_passes=False),
)
def _charge_eq_sc(e_hbm, s_hbm, seg_hbm, out_hbm,
                  ev, sv, sr, zb, idq, e2, s2, sg2, ov, als, ale,
                  acc_s, acc_e, sem_in, sem_sc):
    cid = lax.axis_index("c")
    sid = lax.axis_index("s")
    wid = sid * NC + cid

    start1 = sid * C1
    base1 = jnp.minimum(start1, N - C1)
    base2 = jnp.minimum(wid * C2, N - C2)
    loads = [
        pltpu.async_copy(e_hbm.at[pl.ds(base1, C1)], ev, sem_in),
        pltpu.async_copy(s_hbm.at[pl.ds(base1, C1)], sv, sem_in),
        pltpu.async_copy(seg_hbm.at[pl.ds(base1, C1)], sr.at[pl.ds(0, C1)],
                         sem_in),
        pltpu.async_copy(e_hbm.at[pl.ds(base2, C2)], e2, sem_in),
        pltpu.async_copy(s_hbm.at[pl.ds(base2, C2)], s2, sem_in),
        pltpu.async_copy(seg_hbm.at[pl.ds(base2, C2)], sg2, sem_in),
    ]

    zeros16 = jnp.zeros((L,), jnp.float32)
    for k in range(ZB // L):
        zb[pl.ds(k * L, L)] = zeros16
    pltpu.sync_copy(zb, acc_s.at[pl.ds(sid * ZB, ZB)])
    pltpu.sync_copy(zb, acc_e.at[pl.ds(sid * ZB, ZB)])

    iota16 = lax.iota(jnp.int32, L)

    @plsc.parallel_loop(0, GPAD, step=L, unroll=4)
    def mk_idq(k):
        idq[pl.ds(k, L)] = k + iota16

    plsc.subcore_barrier()

    for d in loads:
        d.wait()

    sr[pl.ds(C1, L)] = jnp.full((L,), G, jnp.int32)

    skip = (start1 - base1) // L

    @pl.when(skip > 0)
    def _():
        for k in range((NS * C1 - N) // L):
            sv[pl.ds(k * L, L)] = jnp.full((L,), jnp.inf, jnp.float32)

    seg_lo = jnp.min(sr[pl.ds(0, L)])
    seg_hi = jnp.max(sr[pl.ds(C1 - L, L)])
    blo = seg_lo // 128
    nblk = seg_hi // 128 - blo + 1

    @plsc.parallel_loop(0, nblk * 128, step=L, unroll=2)
    def zero_blk(k):
        sl = pl.ds(blo * 128 + k, L)
        als[sl] = zeros16
        ale[sl] = zeros16

    lane15 = iota16 == (L - 1)

    @plsc.parallel_loop(0, C1, step=L, unroll=8)
    def p1(i):
        sl = pl.ds(i, L)
        seg = sr[sl]
        nxt = sr[pl.ds(i + 1, L)]
        sinv = 1.0 / sv[sl]
        es = ev[sl] * sinv
        cs_s = plsc.cumsum(sinv)
        cs_e = plsc.cumsum(es)
        bnd = seg != nxt
        pos = jnp.logical_or(bnd, lane15)
        neg = jnp.logical_and(bnd, jnp.logical_not(lane15))
        plsc.addupdate_scatter(als, [seg], cs_s, mask=pos)
        plsc.addupdate_scatter(ale, [seg], cs_e, mask=pos)
        plsc.addupdate_scatter(als, [nxt], -cs_s, mask=neg)
        plsc.addupdate_scatter(ale, [nxt], -cs_e, mask=neg)

    NBLK = GPAD // 128
    for b in range(NBLK):
        @pl.when(jnp.logical_and(b >= blo, b - blo < nblk))
        def _():
            sl = pl.ds(b * 128, 128)
            pltpu.async_copy(als.at[sl], acc_s.at[idq.at[sl]], sem_sc,
                             add=True)
            pltpu.async_copy(ale.at[sl], acc_e.at[idq.at[sl]], sem_sc,
                             add=True)
    for b in range(NBLK):
        @pl.when(jnp.logical_and(b >= blo, b - blo < nblk))
        def _():
            sl = pl.ds(b * 128, 128)
            pltpu.make_async_copy(als.at[sl], acc_s.at[idq.at[sl]],
                                  sem_sc).wait()
            pltpu.make_async_copy(ale.at[sl], acc_e.at[idq.at[sl]],
                                  sem_sc).wait()

    plsc.subcore_barrier()

    wlo = jnp.min(sg2[pl.ds(0, L)]) // 128
    wnblk = jnp.max(sg2[pl.ds(C2 - L, L)]) // 128 - wlo + 1

    for b in range(NBLK):
        @pl.when(jnp.logical_and(b >= wlo, b - wlo < wnblk))
        def _():
            sl = pl.ds(b * 128, 128)
            pltpu.async_copy(acc_s.at[sl], als.at[sl], sem_in)
            pltpu.async_copy(acc_e.at[sl], ale.at[sl], sem_in)
    for b in range(NBLK):
        @pl.when(jnp.logical_and(b >= wlo, b - wlo < wnblk))
        def _():
            sl = pl.ds(b * 128, 128)
            pltpu.make_async_copy(acc_s.at[sl], als.at[sl], sem_in).wait()
            pltpu.make_async_copy(acc_e.at[sl], ale.at[sl], sem_in).wait()

    @plsc.parallel_loop(0, C2, step=L, unroll=8)
    def p2(i):
        sl = pl.ds(i, L)
        sg = sg2[sl]
        gs = plsc.load_gather(als, [sg])
        ge = plsc.load_gather(ale, [sg])
        sinv = 1.0 / s2[sl]
        ov[sl] = sinv * (ge / gs) - e2[sl] * sinv

    pltpu.sync_copy(ov, out_hbm.at[pl.ds(base2, C2)])


def kernel(e, s, segment_ids):
    out = _charge_eq_sc(e.reshape(N), s.reshape(N), segment_ids)
    return out.reshape(N, 1)


# speedup vs baseline: 1.0512x; 1.0512x over previous
import functools

import jax
import jax.numpy as jnp
from jax import lax
from jax.experimental import pallas as pl
from jax.experimental.pallas import tpu as pltpu
from jax.experimental.pallas import tpu_sc as plsc

N = 100000
G = 5000
NC = 2
NS = 16
L = 16

C1 = 6272
C2 = 3136
GPAD = 5120
ZB = GPAD // NS

_mesh = plsc.VectorSubcoreMesh(
    core_axis_name="c", subcore_axis_name="s", num_cores=NC, num_subcores=NS)


@functools.partial(
    pl.kernel,
    out_type=jax.ShapeDtypeStruct((N,), jnp.float32),
    mesh=_mesh,
    scratch_types=[
        pltpu.VMEM((C1,), jnp.float32),
        pltpu.VMEM((C1,), jnp.float32),
        pltpu.VMEM((C1 + L,), jnp.int32),
        pltpu.VMEM((ZB,), jnp.float32),
        pltpu.VMEM((GPAD,), jnp.int32),
        pltpu.VMEM((C2,), jnp.float32),
        pltpu.VMEM((C2,), jnp.float32),
        pltpu.VMEM((C2,), jnp.int32),
        pltpu.VMEM((C2,), jnp.float32),
        pltpu.VMEM((GPAD,), jnp.float32),
        pltpu.VMEM((GPAD,), jnp.float32),
        pltpu.VMEM_SHARED((GPAD,), jnp.float32),
        pltpu.VMEM_SHARED((GPAD,), jnp.float32),
        pltpu.SemaphoreType.DMA,
        pltpu.SemaphoreType.DMA,
    ],
    compiler_params=pltpu.CompilerParams(needs_layout_passes=False),
)
def _charge_eq_sc(e_hbm, s_hbm, seg_hbm, out_hbm,
                  ev, sv, sr, zb, idq, e2, s2, sg2, ov, als, ale,
                  acc_s, acc_e, sem_in, sem_sc):
    cid = lax.axis_index("c")
    sid = lax.axis_index("s")
    wid = sid * NC + cid

    start1 = sid * C1
    base1 = jnp.minimum(start1, N - C1)
    base2 = jnp.minimum(wid * C2, N - C2)
    loads = [
        pltpu.async_copy(e_hbm.at[pl.ds(base1, C1)], ev, sem_in),
        pltpu.async_copy(s_hbm.at[pl.ds(base1, C1)], sv, sem_in),
        pltpu.async_copy(seg_hbm.at[pl.ds(base1, C1)], sr.at[pl.ds(0, C1)],
                         sem_in),
        pltpu.async_copy(e_hbm.at[pl.ds(base2, C2)], e2, sem_in),
        pltpu.async_copy(s_hbm.at[pl.ds(base2, C2)], s2, sem_in),
        pltpu.async_copy(seg_hbm.at[pl.ds(base2, C2)], sg2, sem_in),
    ]

    zeros16 = jnp.zeros((L,), jnp.float32)
    for k in range(ZB // L):
        zb[pl.ds(k * L, L)] = zeros16
    pltpu.sync_copy(zb, acc_s.at[pl.ds(sid * ZB, ZB)])
    pltpu.sync_copy(zb, acc_e.at[pl.ds(sid * ZB, ZB)])

    iota16 = lax.iota(jnp.int32, L)

    @plsc.parallel_loop(0, GPAD, step=L, unroll=4)
    def mk_idq(k):
        idq[pl.ds(k, L)] = k + iota16

    plsc.subcore_barrier()

    for d in loads:
        d.wait()

    sr[pl.ds(C1, L)] = jnp.full((L,), G, jnp.int32)

    skip = (start1 - base1) // L

    @pl.when(skip > 0)
    def _():
        for k in range((NS * C1 - N) // L):
            sv[pl.ds(k * L, L)] = jnp.full((L,), jnp.inf, jnp.float32)

    seg_lo = jnp.min(sr[pl.ds(0, L)])
    seg_hi = jnp.max(sr[pl.ds(C1 - L, L)])
    blo = seg_lo // 128
    nblk = seg_hi // 128 - blo + 1

    @plsc.parallel_loop(0, nblk * 128, step=L, unroll=2)
    def zero_blk(k):
        sl = pl.ds(blo * 128 + k, L)
        als[sl] = zeros16
        ale[sl] = zeros16

    lane15 = iota16 == (L - 1)

    @plsc.parallel_loop(0, C1, step=L, unroll=8)
    def p1(i):
        sl = pl.ds(i, L)
        seg = sr[sl]
        nxt = sr[pl.ds(i + 1, L)]
        sinv = 1.0 / sv[sl]
        es = ev[sl] * sinv
        cs_s = plsc.cumsum(sinv)
        cs_e = plsc.cumsum(es)
        bnd = seg != nxt
        pos = jnp.logical_or(bnd, lane15)
        neg = jnp.logical_and(bnd, jnp.logical_not(lane15))
        plsc.addupdate_scatter(als, [seg], cs_s, mask=pos)
        plsc.addupdate_scatter(ale, [seg], cs_e, mask=pos)
        plsc.addupdate_scatter(als, [nxt], -cs_s, mask=neg)
        plsc.addupdate_scatter(ale, [nxt], -cs_e, mask=neg)

    def comb(b, carry):
        sl = pl.ds((blo + b) * 128, 128)
        d1 = pltpu.async_copy(als.at[sl], acc_s.at[idq.at[sl]], sem_sc,
                              add=True)
        d2 = pltpu.async_copy(ale.at[sl], acc_e.at[idq.at[sl]], sem_sc,
                              add=True)
        d1.wait()
        d2.wait()
        return carry

    lax.fori_loop(0, nblk, comb, 0)

    plsc.subcore_barrier()

    wlo = jnp.min(sg2[pl.ds(0, L)]) // 128
    wnblk = jnp.max(sg2[pl.ds(C2 - L, L)]) // 128 - wlo + 1

    def pull(b, carry):
        sl = pl.ds((wlo + b) * 128, 128)
        d1 = pltpu.async_copy(acc_s.at[sl], als.at[sl], sem_in)
        d2 = pltpu.async_copy(acc_e.at[sl], ale.at[sl], sem_in)
        d1.wait()
        d2.wait()
        return carry

    lax.fori_loop(0, wnblk, pull, 0)

    @plsc.parallel_loop(0, C2, step=L, unroll=8)
    def p2(i):
        sl = pl.ds(i, L)
        sg = sg2[sl]
        gs = plsc.load_gather(als, [sg])
        ge = plsc.load_gather(ale, [sg])
        sinv = 1.0 / s2[sl]
        ov[sl] = sinv * (ge / gs) - e2[sl] * sinv

    pltpu.sync_copy(ov, out_hbm.at[pl.ds(base2, C2)])


def kernel(e, s, segment_ids):
    out = _charge_eq_sc(e.reshape(N), s.reshape(N), segment_ids)
    return out.reshape(N, 1)


# speedup vs baseline: 1.0975x; 1.0440x over previous
import functools

import jax
import jax.numpy as jnp
from jax import lax
from jax.experimental import pallas as pl
from jax.experimental.pallas import tpu as pltpu
from jax.experimental.pallas import tpu_sc as plsc

N = 100000
G = 5000
NC = 1
NS = 16
L = 16

C1 = 6272
C2 = 6272
GPAD = 5120
ZB = GPAD // NS

_mesh = plsc.VectorSubcoreMesh(
    core_axis_name="c", subcore_axis_name="s", num_cores=NC, num_subcores=NS)


@functools.partial(
    pl.kernel,
    out_type=jax.ShapeDtypeStruct((N,), jnp.float32),
    mesh=_mesh,
    scratch_types=[
        pltpu.VMEM((C1,), jnp.float32),
        pltpu.VMEM((C1,), jnp.float32),
        pltpu.VMEM((C1 + L,), jnp.int32),
        pltpu.VMEM((ZB,), jnp.float32),
        pltpu.VMEM((GPAD,), jnp.int32),
        pltpu.VMEM((C2,), jnp.float32),
        pltpu.VMEM((C2,), jnp.float32),
        pltpu.VMEM((C2,), jnp.int32),
        pltpu.VMEM((C2,), jnp.float32),
        pltpu.VMEM((GPAD,), jnp.float32),
        pltpu.VMEM((GPAD,), jnp.float32),
        pltpu.VMEM_SHARED((GPAD,), jnp.float32),
        pltpu.VMEM_SHARED((GPAD,), jnp.float32),
        pltpu.SemaphoreType.DMA,
        pltpu.SemaphoreType.DMA,
    ],
    compiler_params=pltpu.CompilerParams(needs_layout_passes=False),
)
def _charge_eq_sc(e_hbm, s_hbm, seg_hbm, out_hbm,
                  ev, sv, sr, zb, idq, e2, s2, sg2, ov, als, ale,
                  acc_s, acc_e, sem_in, sem_sc):
    cid = lax.axis_index("c")
    sid = lax.axis_index("s")
    wid = sid * NC + cid

    start1 = sid * C1
    base1 = jnp.minimum(start1, N - C1)
    base2 = jnp.minimum(wid * C2, N - C2)
    loads = [
        pltpu.async_copy(e_hbm.at[pl.ds(base1, C1)], ev, sem_in),
        pltpu.async_copy(s_hbm.at[pl.ds(base1, C1)], sv, sem_in),
        pltpu.async_copy(seg_hbm.at[pl.ds(base1, C1)], sr.at[pl.ds(0, C1)],
                         sem_in),
        pltpu.async_copy(e_hbm.at[pl.ds(base2, C2)], e2, sem_in),
        pltpu.async_copy(s_hbm.at[pl.ds(base2, C2)], s2, sem_in),
        pltpu.async_copy(seg_hbm.at[pl.ds(base2, C2)], sg2, sem_in),
    ]

    zeros16 = jnp.zeros((L,), jnp.float32)
    for k in range(ZB // L):
        zb[pl.ds(k * L, L)] = zeros16
    pltpu.sync_copy(zb, acc_s.at[pl.ds(sid * ZB, ZB)])
    pltpu.sync_copy(zb, acc_e.at[pl.ds(sid * ZB, ZB)])

    iota16 = lax.iota(jnp.int32, L)

    @plsc.parallel_loop(0, GPAD, step=L, unroll=4)
    def mk_idq(k):
        idq[pl.ds(k, L)] = k + iota16

    plsc.subcore_barrier()

    for d in loads:
        d.wait()

    sr[pl.ds(C1, L)] = jnp.full((L,), G, jnp.int32)

    skip = (start1 - base1) // L

    @pl.when(skip > 0)
    def _():
        for k in range((NS * C1 - N) // L):
            sv[pl.ds(k * L, L)] = jnp.full((L,), jnp.inf, jnp.float32)

    seg_lo = jnp.min(sr[pl.ds(0, L)])
    seg_hi = jnp.max(sr[pl.ds(C1 - L, L)])
    blo = seg_lo // 128
    nblk = seg_hi // 128 - blo + 1

    @plsc.parallel_loop(0, nblk * 128, step=L, unroll=2)
    def zero_blk(k):
        sl = pl.ds(blo * 128 + k, L)
        als[sl] = zeros16
        ale[sl] = zeros16

    lane15 = iota16 == (L - 1)

    @plsc.parallel_loop(0, C1, step=L, unroll=8)
    def p1(i):
        sl = pl.ds(i, L)
        seg = sr[sl]
        nxt = sr[pl.ds(i + 1, L)]
        sinv = 1.0 / sv[sl]
        es = ev[sl] * sinv
        cs_s = plsc.cumsum(sinv)
        cs_e = plsc.cumsum(es)
        bnd = seg != nxt
        pos = jnp.logical_or(bnd, lane15)
        neg = jnp.logical_and(bnd, jnp.logical_not(lane15))
        plsc.addupdate_scatter(als, [seg], cs_s, mask=pos)
        plsc.addupdate_scatter(ale, [seg], cs_e, mask=pos)
        plsc.addupdate_scatter(als, [nxt], -cs_s, mask=neg)
        plsc.addupdate_scatter(ale, [nxt], -cs_e, mask=neg)

    def comb(b, carry):
        sl = pl.ds((blo + b) * 128, 128)
        d1 = pltpu.async_copy(als.at[sl], acc_s.at[idq.at[sl]], sem_sc,
                              add=True)
        d2 = pltpu.async_copy(ale.at[sl], acc_e.at[idq.at[sl]], sem_sc,
                              add=True)
        d1.wait()
        d2.wait()
        return carry

    lax.fori_loop(0, nblk, comb, 0)

    plsc.subcore_barrier()

    wlo = jnp.min(sg2[pl.ds(0, L)]) // 128
    wnblk = jnp.max(sg2[pl.ds(C2 - L, L)]) // 128 - wlo + 1

    def pull(b, carry):
        sl = pl.ds((wlo + b) * 128, 128)
        d1 = pltpu.async_copy(acc_s.at[sl], als.at[sl], sem_in)
        d2 = pltpu.async_copy(acc_e.at[sl], ale.at[sl], sem_in)
        d1.wait()
        d2.wait()
        return carry

    lax.fori_loop(0, wnblk, pull, 0)

    @plsc.parallel_loop(0, C2, step=L, unroll=8)
    def p2(i):
        sl = pl.ds(i, L)
        sg = sg2[sl]
        gs = plsc.load_gather(als, [sg])
        ge = plsc.load_gather(ale, [sg])
        sinv = 1.0 / s2[sl]
        ov[sl] = sinv * (ge / gs) - e2[sl] * sinv

    pltpu.sync_copy(ov, out_hbm.at[pl.ds(base2, C2)])


def kernel(e, s, segment_ids):
    out = _charge_eq_sc(e.reshape(N), s.reshape(N), segment_ids)
    return out.reshape(N, 1)


# speedup vs baseline: 1.1146x; 1.0156x over previous
import functools

import jax
import jax.numpy as jnp
from jax import lax
from jax.experimental import pallas as pl
from jax.experimental.pallas import tpu as pltpu
from jax.experimental.pallas import tpu_sc as plsc

N = 100000
G = 5000
NC = 1
NS = 16
L = 16

C1 = 6272
C2 = 6272
GPAD = 5120
ZB = GPAD // NS

_mesh = plsc.VectorSubcoreMesh(
    core_axis_name="c", subcore_axis_name="s", num_cores=NC, num_subcores=NS)


@functools.partial(
    pl.kernel,
    out_type=jax.ShapeDtypeStruct((N,), jnp.float32),
    mesh=_mesh,
    scratch_types=[
        pltpu.VMEM((C1,), jnp.float32),
        pltpu.VMEM((C1,), jnp.float32),
        pltpu.VMEM((C1 + L,), jnp.int32),
        pltpu.VMEM((ZB,), jnp.float32),
        pltpu.VMEM((GPAD,), jnp.int32),
        pltpu.VMEM((C2,), jnp.float32),
        pltpu.VMEM((GPAD,), jnp.float32),
        pltpu.VMEM((GPAD,), jnp.float32),
        pltpu.VMEM_SHARED((GPAD,), jnp.float32),
        pltpu.VMEM_SHARED((GPAD,), jnp.float32),
        pltpu.SemaphoreType.DMA,
        pltpu.SemaphoreType.DMA,
    ],
    compiler_params=pltpu.CompilerParams(needs_layout_passes=False),
)
def _charge_eq_sc(e_hbm, s_hbm, seg_hbm, out_hbm,
                  ev, sv, sr, zb, idq, ov, als, ale,
                  acc_s, acc_e, sem_in, sem_sc):
    sid = lax.axis_index("s")

    start1 = sid * C1
    base1 = jnp.minimum(start1, N - C1)
    loads = [
        pltpu.async_copy(e_hbm.at[pl.ds(base1, C1)], ev, sem_in),
        pltpu.async_copy(s_hbm.at[pl.ds(base1, C1)], sv, sem_in),
        pltpu.async_copy(seg_hbm.at[pl.ds(base1, C1)], sr.at[pl.ds(0, C1)],
                         sem_in),
    ]

    zeros16 = jnp.zeros((L,), jnp.float32)
    for k in range(ZB // L):
        zb[pl.ds(k * L, L)] = zeros16
    pltpu.sync_copy(zb, acc_s.at[pl.ds(sid * ZB, ZB)])
    pltpu.sync_copy(zb, acc_e.at[pl.ds(sid * ZB, ZB)])

    iota16 = lax.iota(jnp.int32, L)

    @plsc.parallel_loop(0, GPAD, step=L, unroll=4)
    def mk_idq(k):
        idq[pl.ds(k, L)] = k + iota16

    plsc.subcore_barrier()

    for d in loads:
        d.wait()

    sr[pl.ds(C1, L)] = jnp.full((L,), G, jnp.int32)

    skip = (start1 - base1) // L

    @pl.when(skip > 0)
    def _():
        for k in range((NS * C1 - N) // L):
            sv[pl.ds(k * L, L)] = jnp.full((L,), jnp.inf, jnp.float32)

    seg_lo = jnp.min(sr[pl.ds(0, L)])
    seg_hi = jnp.max(sr[pl.ds(C1 - L, L)])
    blo = seg_lo // 128
    nblk = seg_hi // 128 - blo + 1

    @plsc.parallel_loop(0, nblk * 128, step=L, unroll=2)
    def zero_blk(k):
        sl = pl.ds(blo * 128 + k, L)
        als[sl] = zeros16
        ale[sl] = zeros16

    lane15 = iota16 == (L - 1)

    @plsc.parallel_loop(0, C1, step=L, unroll=8)
    def p1(i):
        sl = pl.ds(i, L)
        seg = sr[sl]
        nxt = sr[pl.ds(i + 1, L)]
        sinv = 1.0 / sv[sl]
        es = ev[sl] * sinv
        cs_s = plsc.cumsum(sinv)
        cs_e = plsc.cumsum(es)
        bnd = seg != nxt
        pos = jnp.logical_or(bnd, lane15)
        neg = jnp.logical_and(bnd, jnp.logical_not(lane15))
        plsc.addupdate_scatter(als, [seg], cs_s, mask=pos)
        plsc.addupdate_scatter(ale, [seg], cs_e, mask=pos)
        plsc.addupdate_scatter(als, [nxt], -cs_s, mask=neg)
        plsc.addupdate_scatter(ale, [nxt], -cs_e, mask=neg)

    @pl.when(skip > 0)
    def _():
        pltpu.sync_copy(s_hbm.at[pl.ds(base1, NS * C1 - N)],
                        sv.at[pl.ds(0, NS * C1 - N)])

    def comb(b, carry):
        sl = pl.ds((blo + b) * 128, 128)
        d1 = pltpu.async_copy(als.at[sl], acc_s.at[idq.at[sl]], sem_sc,
                              add=True)
        d2 = pltpu.async_copy(ale.at[sl], acc_e.at[idq.at[sl]], sem_sc,
                              add=True)
        d1.wait()
        d2.wait()
        return carry

    lax.fori_loop(0, nblk, comb, 0)

    plsc.subcore_barrier()

    def pull(b, carry):
        sl = pl.ds((blo + b) * 128, 128)
        d1 = pltpu.async_copy(acc_s.at[sl], als.at[sl], sem_in)
        d2 = pltpu.async_copy(acc_e.at[sl], ale.at[sl], sem_in)
        d1.wait()
        d2.wait()
        return carry

    lax.fori_loop(0, nblk, pull, 0)

    @plsc.parallel_loop(0, C2, step=L, unroll=8)
    def p2(i):
        sl = pl.ds(i, L)
        sg = sr[sl]
        gs = plsc.load_gather(als, [sg])
        ge = plsc.load_gather(ale, [sg])
        sinv = 1.0 / sv[sl]
        ov[sl] = sinv * (ge / gs) - ev[sl] * sinv

    pltpu.sync_copy(ov, out_hbm.at[pl.ds(base1, C2)])


def kernel(e, s, segment_ids):
    out = _charge_eq_sc(e.reshape(N), s.reshape(N), segment_ids)
    return out.reshape(N, 1)
